# baseline (speedup 1.0000x reference)
import jax
import jax.numpy as jnp
from jax import lax
from jax.experimental import pallas as pl
from jax.experimental.pallas import tpu as pltpu, tpu_sc as plsc

_NC = 2
_NS = 16


def _sc_lookup_body(pid_hbm, emb_hbm, out_hbm, pid_s):
    cid = lax.axis_index("c")

    @pl.when(cid == 0)
    def _():
        pltpu.sync_copy(pid_hbm, pid_s)
        p = pid_s[0]
        pltpu.sync_copy(emb_hbm.at[pl.ds(p * 128, 128)], out_hbm)


def _sc_lookup(pid, emb_flat):
    d = 128
    mesh = plsc.ScalarSubcoreMesh(axis_name="c", num_cores=_NC)
    f = pl.kernel(
        _sc_lookup_body,
        out_type=jax.ShapeDtypeStruct((d,), emb_flat.dtype),
        mesh=mesh,
        scratch_types=[
            pltpu.SMEM((1,), jnp.int32),
        ],
    )
    return f(pid, emb_flat)


def _tc_add_body(x_ref, v_ref, o_ref):
    o_ref[...] = x_ref[...] + v_ref[...]


def _tc_add(x, pert_vec):
    n, d = x.shape
    blk = 25000
    return pl.pallas_call(
        _tc_add_body,
        grid=(n // blk,),
        in_specs=[
            pl.BlockSpec((blk, d), lambda i: (i, 0)),
            pl.BlockSpec((1, d), lambda i: (0, 0)),
        ],
        out_specs=pl.BlockSpec((blk, d), lambda i: (i, 0)),
        out_shape=jax.ShapeDtypeStruct((n, d), x.dtype),
        compiler_params=pltpu.CompilerParams(
            dimension_semantics=("arbitrary",),
        ),
    )(x, pert_vec)


def kernel(x, pert_id, embedding_weight):
    pid = jnp.reshape(pert_id, (-1,))[0:1].astype(jnp.int32)
    emb_flat = embedding_weight.reshape(-1)
    pert_vec = _sc_lookup(pid, emb_flat).reshape(1, -1)
    return _tc_add(x, pert_vec)
